# baseline (speedup 1.0000x reference)
import functools

import jax
import jax.numpy as jnp
from jax import lax
from jax.experimental import pallas as pl
from jax.experimental.pallas import tpu as pltpu
from jax.experimental.pallas import tpu_sc as plsc

B, C, H, W = 8, 96, 384, 384
HW = H * W
SL = HW // 128
K = HW // 2
CB_MEAN = 8
TS_APPLY = 48
I32MIN = -(2**31)
NQ = 4
Q = HW // NQ
NVQ = Q // 16


def _mean_body(x_ref, o_ref):
    c = pl.program_id(1)

    @pl.when(c == 0)
    def _init():
        o_ref[...] = jnp.zeros_like(o_ref)

    xm = x_ref[0].reshape(CB_MEAN, SL * 128)
    ones = jnp.ones((1, CB_MEAN), jnp.float32)
    s = jax.lax.dot_general(ones, xm, (((1,), (0,)), ((), ())),
                            preferred_element_type=jnp.float32)
    o_ref[...] += s.reshape(1, SL, 128)

    @pl.when(c == (C // CB_MEAN) - 1)
    def _fin():
        o_ref[...] *= jnp.float32(1.0 / C)


def _lsr(x, n):
    return lax.shift_right_logical(x, jnp.int32(n) if isinstance(n, int) else n)


def _sc_select_body(e_hbm, m_hbm, ebuf, kbuf, hloc, hcmb, t4, shr):
    c = lax.axis_index("c")
    s = lax.axis_index("s")
    row = c * 4 + s // 4
    q = s % 4
    g = s - q
    base = row * HW + q * Q
    imin = jnp.int32(I32MIN)
    ones = jnp.ones((16,), jnp.int32)
    lane = jax.lax.broadcasted_iota(jnp.int32, (16,), 0)

    def dsv(i):
        return pl.ds(pl.multiple_of(i * 16, 16), 16)

    pltpu.sync_copy(e_hbm.at[pl.ds(base, Q)], ebuf)

    def keys_init(i, _):
        u = lax.bitcast_convert_type(ebuf[dsv(i)], jnp.int32)
        kbuf[dsv(i)] = jnp.where(u >= 0, u, imin - u)
        return 0

    lax.fori_loop(0, NVQ, keys_init, 0)

    def hist_pass(nbins, bin_fn, match_fn):
        def zero(i, _):
            hloc[dsv(i)] = jnp.zeros((16,), jnp.int32)
            return 0

        lax.fori_loop(0, nbins // 16, zero, 0)

        def acc(i, _):
            b = kbuf[dsv(i)] ^ imin
            if match_fn is None:
                plsc.addupdate_scatter(hloc, [bin_fn(b)], ones)
            else:
                plsc.addupdate_scatter(hloc, [bin_fn(b)], ones,
                                       mask=match_fn(b))
            return 0

        lax.fori_loop(0, NVQ, acc, 0)

    def combine(p, nbins):
        pltpu.sync_copy(hloc.at[pl.ds(0, nbins)], shr.at[s, p, pl.ds(0, nbins)])
        plsc.subcore_barrier()
        for t in range(4):
            pltpu.sync_copy(shr.at[g + t, p, pl.ds(0, nbins)],
                            t4.at[t, pl.ds(0, nbins)])

        def csum(i, _):
            hcmb[dsv(i)] = (t4[0, dsv(i)] + t4[1, dsv(i)]
                            + t4[2, dsv(i)] + t4[3, dsv(i)])
            return 0

        lax.fori_loop(0, nbins // 16, csum, 0)

    def scan(nbins, target):
        def body(i, carry):
            run, p, c_at, h_at = carry
            h16 = hcmb[dsv(i)]
            cum = plsc.cumsum(h16) + run
            cross = cum > target
            npop = jnp.max(plsc.all_reduce_population_count(cross))
            ffs = jnp.max(plsc.all_reduce_ffs(cross))
            oh = lane == ffs
            cv = jnp.max(jnp.where(oh, cum, 0))
            hv = jnp.max(jnp.where(oh, h16, 0))
            isnew = (p < 0) & (npop > 0)
            p = jnp.where(isnew, i * 16 + ffs, p)
            c_at = jnp.where(isnew, cv, c_at)
            h_at = jnp.where(isnew, hv, h_at)
            return jnp.max(cum), p, c_at, h_at

        _, p, c_at, h_at = lax.fori_loop(
            0, nbins // 16, body,
            (jnp.int32(0), jnp.int32(-1), jnp.int32(0), jnp.int32(0)))
        return p, c_at, h_at

    hist_pass(4096, lambda b: _lsr(b, 20), None)
    combine(0, 4096)
    p1, c1, h1 = scan(4096, jnp.int32(HW - K))
    k2 = jnp.int32(K) - (jnp.int32(HW) - c1)

    hist_pass(4096, lambda b: _lsr(b, 8) & 0xFFF,
              lambda b: _lsr(b, 20) == p1)
    combine(1, 4096)
    p2, c2, h2 = scan(4096, h1 - k2)
    k3 = k2 - (h1 - c2)

    hist_pass(256, lambda b: b & 0xFF,
              lambda b: _lsr(b, 8) == ((p1 << 12) | p2))
    combine(2, 256)
    p3, c3, h3 = scan(256, h2 - k3)
    k4 = k3 - (h2 - c3)

    t_key = ((p1 << 20) | (p2 << 8) | p3) ^ imin

    my16 = hloc[pl.ds(pl.multiple_of((p3 // 16) * 16, 16), 16)]
    myeq = jnp.max(jnp.where(lane == (p3 % 16), my16, 0))
    hloc[pl.ds(0, 16)] = jnp.broadcast_to(myeq, (16,)).astype(jnp.int32)
    pltpu.sync_copy(hloc.at[pl.ds(0, 16)], shr.at[s, 3, pl.ds(0, 16)])
    plsc.subcore_barrier()
    neq_before = jnp.int32(0)
    for t in range(3):
        pltpu.sync_copy(shr.at[g + t, 3, pl.ds(0, 16)], t4.at[t, pl.ds(0, 16)])
        cnt_t = jnp.max(t4[t, pl.ds(0, 16)])
        neq_before += jnp.where(jnp.int32(t) < q, cnt_t, 0)
    need_local = k4 - neq_before

    def mask_build(i, r):
        key = kbuf[dsv(i)]
        gt = key > t_key
        eq = key == t_key
        ec = jnp.where(eq, 1, 0).astype(jnp.int32)
        cum = plsc.cumsum(ec)
        take = eq & ((r + cum - ec) < need_local)
        ebuf[dsv(i)] = jnp.where(gt | take, 1.0, 0.0).astype(jnp.float32)
        return r + jnp.max(cum)

    lax.fori_loop(0, NVQ, mask_build, jnp.int32(0))
    pltpu.sync_copy(ebuf, m_hbm.at[pl.ds(base, Q)])


_sc_select = functools.partial(
    pl.kernel,
    out_type=jax.ShapeDtypeStruct((B * HW,), jnp.float32),
    mesh=plsc.VectorSubcoreMesh(core_axis_name="c", subcore_axis_name="s",
                                num_cores=2, num_subcores=16),
    compiler_params=pltpu.CompilerParams(needs_layout_passes=False),
    scratch_types=[
        pltpu.VMEM((Q,), jnp.float32),
        pltpu.VMEM((Q,), jnp.int32),
        pltpu.VMEM((4096,), jnp.int32),
        pltpu.VMEM((4096,), jnp.int32),
        pltpu.VMEM((4, 4096), jnp.int32),
        pltpu.VMEM_SHARED((16, 4, 4096), jnp.int32),
    ],
)(_sc_select_body)


def _apply_body(x_ref, m_ref, o_ref):
    o_ref[...] = x_ref[...] * m_ref[...][:, None]


@jax.jit
def kernel(x):
    xr = x.reshape(B, C, SL, 128)

    energy = pl.pallas_call(
        _mean_body,
        grid=(B, C // CB_MEAN),
        in_specs=[pl.BlockSpec((1, CB_MEAN, SL, 128), lambda b, c: (b, c, 0, 0))],
        out_specs=pl.BlockSpec((1, SL, 128), lambda b, c: (b, 0, 0)),
        out_shape=jax.ShapeDtypeStruct((B, SL, 128), jnp.float32),
    )(xr)

    mask = _sc_select(energy.reshape(B * HW)).reshape(B, SL, 128)

    out = pl.pallas_call(
        _apply_body,
        grid=(B, SL // TS_APPLY),
        in_specs=[
            pl.BlockSpec((1, C, TS_APPLY, 128), lambda b, j: (b, 0, j, 0)),
            pl.BlockSpec((1, TS_APPLY, 128), lambda b, j: (b, j, 0)),
        ],
        out_specs=pl.BlockSpec((1, C, TS_APPLY, 128), lambda b, j: (b, 0, j, 0)),
        out_shape=jax.ShapeDtypeStruct((B, C, SL, 128), jnp.float32),
    )(xr, mask)

    return out.reshape(B, C, H, W)


# speedup vs baseline: 1.0457x; 1.0457x over previous
import functools

import jax
import jax.numpy as jnp
from jax import lax
from jax.experimental import pallas as pl
from jax.experimental.pallas import tpu as pltpu
from jax.experimental.pallas import tpu_sc as plsc

B, C, H, W = 8, 96, 384, 384
HW = H * W
SL = HW // 128
K = HW // 2
CB_MEAN = 8
TS_APPLY = 96
I32MIN = -(2**31)
NQ = 4
Q = HW // NQ
NVQ = Q // 16


def _mean_body(x_ref, o_ref):
    c = pl.program_id(1)

    @pl.when(c == 0)
    def _init():
        o_ref[...] = jnp.zeros_like(o_ref)

    o_ref[...] += jnp.sum(x_ref[...], axis=1)

    @pl.when(c == (C // CB_MEAN) - 1)
    def _fin():
        o_ref[...] *= jnp.float32(1.0 / C)


def _lsr(x, n):
    return lax.shift_right_logical(x, jnp.int32(n) if isinstance(n, int) else n)


def _sc_select_body(e_hbm, m_hbm, ebuf, kbuf, hloc, hcmb, t4, shr):
    c = lax.axis_index("c")
    s = lax.axis_index("s")
    row = c * 4 + s // 4
    q = s % 4
    g = s - q
    base = row * HW + q * Q
    imin = jnp.int32(I32MIN)
    ones = jnp.ones((16,), jnp.int32)
    lane = jax.lax.broadcasted_iota(jnp.int32, (16,), 0)

    def dsv(i):
        return pl.ds(pl.multiple_of(i * 16, 16), 16)

    pltpu.sync_copy(e_hbm.at[pl.ds(base, Q)], ebuf)

    def keys_init(i, _):
        u = lax.bitcast_convert_type(ebuf[dsv(i)], jnp.int32)
        kbuf[dsv(i)] = jnp.where(u >= 0, u, imin - u)
        return 0

    lax.fori_loop(0, NVQ, keys_init, 0)

    def hist_pass(nbins, bin_fn, match_fn):
        def zero(i, _):
            hloc[dsv(i)] = jnp.zeros((16,), jnp.int32)
            return 0

        lax.fori_loop(0, nbins // 16, zero, 0)

        def acc(i, _):
            b = kbuf[dsv(i)] ^ imin
            if match_fn is None:
                plsc.addupdate_scatter(hloc, [bin_fn(b)], ones)
            else:
                plsc.addupdate_scatter(hloc, [bin_fn(b)], ones,
                                       mask=match_fn(b))
            return 0

        lax.fori_loop(0, NVQ, acc, 0)

    def combine(p, nbins):
        pltpu.sync_copy(hloc.at[pl.ds(0, nbins)], shr.at[s, p, pl.ds(0, nbins)])
        plsc.subcore_barrier()
        for t in range(4):
            pltpu.sync_copy(shr.at[g + t, p, pl.ds(0, nbins)],
                            t4.at[t, pl.ds(0, nbins)])

        def csum(i, _):
            hcmb[dsv(i)] = (t4[0, dsv(i)] + t4[1, dsv(i)]
                            + t4[2, dsv(i)] + t4[3, dsv(i)])
            return 0

        lax.fori_loop(0, nbins // 16, csum, 0)

    def scan(nbins, target):
        def body(i, carry):
            run, p, c_at, h_at = carry
            h16 = hcmb[dsv(i)]
            cum = plsc.cumsum(h16) + run
            cross = cum > target
            npop = jnp.max(plsc.all_reduce_population_count(cross))
            ffs = jnp.max(plsc.all_reduce_ffs(cross))
            oh = lane == ffs
            cv = jnp.max(jnp.where(oh, cum, 0))
            hv = jnp.max(jnp.where(oh, h16, 0))
            isnew = (p < 0) & (npop > 0)
            p = jnp.where(isnew, i * 16 + ffs, p)
            c_at = jnp.where(isnew, cv, c_at)
            h_at = jnp.where(isnew, hv, h_at)
            return jnp.max(cum), p, c_at, h_at

        _, p, c_at, h_at = lax.fori_loop(
            0, nbins // 16, body,
            (jnp.int32(0), jnp.int32(-1), jnp.int32(0), jnp.int32(0)))
        return p, c_at, h_at

    hist_pass(4096, lambda b: _lsr(b, 20), None)
    combine(0, 4096)
    p1, c1, h1 = scan(4096, jnp.int32(HW - K))
    k2 = jnp.int32(K) - (jnp.int32(HW) - c1)

    hist_pass(4096, lambda b: _lsr(b, 8) & 0xFFF,
              lambda b: _lsr(b, 20) == p1)
    combine(1, 4096)
    p2, c2, h2 = scan(4096, h1 - k2)
    k3 = k2 - (h1 - c2)

    hist_pass(256, lambda b: b & 0xFF,
              lambda b: _lsr(b, 8) == ((p1 << 12) | p2))
    combine(2, 256)
    p3, c3, h3 = scan(256, h2 - k3)
    k4 = k3 - (h2 - c3)

    t_key = ((p1 << 20) | (p2 << 8) | p3) ^ imin

    my16 = hloc[pl.ds(pl.multiple_of((p3 // 16) * 16, 16), 16)]
    myeq = jnp.max(jnp.where(lane == (p3 % 16), my16, 0))
    hloc[pl.ds(0, 16)] = jnp.broadcast_to(myeq, (16,)).astype(jnp.int32)
    pltpu.sync_copy(hloc.at[pl.ds(0, 16)], shr.at[s, 3, pl.ds(0, 16)])
    plsc.subcore_barrier()
    neq_before = jnp.int32(0)
    for t in range(3):
        pltpu.sync_copy(shr.at[g + t, 3, pl.ds(0, 16)], t4.at[t, pl.ds(0, 16)])
        cnt_t = jnp.max(t4[t, pl.ds(0, 16)])
        neq_before += jnp.where(jnp.int32(t) < q, cnt_t, 0)
    need_local = k4 - neq_before

    def mask_build(i, r):
        key = kbuf[dsv(i)]
        gt = key > t_key
        eq = key == t_key
        ec = jnp.where(eq, 1, 0).astype(jnp.int32)
        cum = plsc.cumsum(ec)
        take = eq & ((r + cum - ec) < need_local)
        ebuf[dsv(i)] = jnp.where(gt | take, 1.0, 0.0).astype(jnp.float32)
        return r + jnp.max(cum)

    lax.fori_loop(0, NVQ, mask_build, jnp.int32(0))
    pltpu.sync_copy(ebuf, m_hbm.at[pl.ds(base, Q)])


_sc_select = functools.partial(
    pl.kernel,
    out_type=jax.ShapeDtypeStruct((B * HW,), jnp.float32),
    mesh=plsc.VectorSubcoreMesh(core_axis_name="c", subcore_axis_name="s",
                                num_cores=2, num_subcores=16),
    compiler_params=pltpu.CompilerParams(needs_layout_passes=False),
    scratch_types=[
        pltpu.VMEM((Q,), jnp.float32),
        pltpu.VMEM((Q,), jnp.int32),
        pltpu.VMEM((4096,), jnp.int32),
        pltpu.VMEM((4096,), jnp.int32),
        pltpu.VMEM((4, 4096), jnp.int32),
        pltpu.VMEM_SHARED((16, 4, 4096), jnp.int32),
    ],
)(_sc_select_body)


def _apply_body(x_ref, m_ref, o_ref):
    o_ref[...] = x_ref[...] * m_ref[...][:, None]


@jax.jit
def kernel(x):
    xr = x.reshape(B, C, SL, 128)

    energy = pl.pallas_call(
        _mean_body,
        grid=(B, C // CB_MEAN),
        in_specs=[pl.BlockSpec((1, CB_MEAN, SL, 128), lambda b, c: (b, c, 0, 0))],
        out_specs=pl.BlockSpec((1, SL, 128), lambda b, c: (b, 0, 0)),
        out_shape=jax.ShapeDtypeStruct((B, SL, 128), jnp.float32),
    )(xr)

    mask = _sc_select(energy.reshape(B * HW)).reshape(B, SL, 128)

    out = pl.pallas_call(
        _apply_body,
        grid=(B, SL // TS_APPLY),
        in_specs=[
            pl.BlockSpec((1, C, TS_APPLY, 128), lambda b, j: (b, 0, j, 0)),
            pl.BlockSpec((1, TS_APPLY, 128), lambda b, j: (b, j, 0)),
        ],
        out_specs=pl.BlockSpec((1, C, TS_APPLY, 128), lambda b, j: (b, 0, j, 0)),
        out_shape=jax.ShapeDtypeStruct((B, C, SL, 128), jnp.float32),
    )(xr, mask)

    return out.reshape(B, C, H, W)


# speedup vs baseline: 1.0475x; 1.0017x over previous
import functools

import jax
import jax.numpy as jnp
from jax import lax
from jax.experimental import pallas as pl
from jax.experimental.pallas import tpu as pltpu
from jax.experimental.pallas import tpu_sc as plsc

B, C, H, W = 8, 96, 384, 384
HW = H * W
SL = HW // 128
K = HW // 2
CB_MEAN = 8
TS_APPLY = 192
I32MIN = -(2**31)
NQ = 4
Q = HW // NQ
NVQ = Q // 16


def _mean_body(x_ref, o_ref):
    c = pl.program_id(1)

    @pl.when(c == 0)
    def _init():
        o_ref[...] = jnp.zeros_like(o_ref)

    o_ref[...] += jnp.sum(x_ref[...], axis=1)

    @pl.when(c == (C // CB_MEAN) - 1)
    def _fin():
        o_ref[...] *= jnp.float32(1.0 / C)


def _lsr(x, n):
    return lax.shift_right_logical(x, jnp.int32(n) if isinstance(n, int) else n)


def _sc_select_body(e_hbm, m_hbm, ebuf, kbuf, hloc, hcmb, t4, shr):
    c = lax.axis_index("c")
    s = lax.axis_index("s")
    row = c * 4 + s // 4
    q = s % 4
    g = s - q
    base = row * HW + q * Q
    imin = jnp.int32(I32MIN)
    ones = jnp.ones((16,), jnp.int32)
    lane = jax.lax.broadcasted_iota(jnp.int32, (16,), 0)

    def dsv(i):
        return pl.ds(pl.multiple_of(i * 16, 16), 16)

    pltpu.sync_copy(e_hbm.at[pl.ds(base, Q)], ebuf)

    def keys_init(i, _):
        u = lax.bitcast_convert_type(ebuf[dsv(i)], jnp.int32)
        kbuf[dsv(i)] = jnp.where(u >= 0, u, imin - u)
        return 0

    lax.fori_loop(0, NVQ, keys_init, 0)

    def hist_pass(nbins, bin_fn, match_fn):
        def zero(i, _):
            hloc[dsv(i)] = jnp.zeros((16,), jnp.int32)
            return 0

        lax.fori_loop(0, nbins // 16, zero, 0)

        def acc(i, _):
            b = kbuf[dsv(i)] ^ imin
            if match_fn is None:
                plsc.addupdate_scatter(hloc, [bin_fn(b)], ones)
            else:
                plsc.addupdate_scatter(hloc, [bin_fn(b)], ones,
                                       mask=match_fn(b))
            return 0

        lax.fori_loop(0, NVQ, acc, 0)

    def combine(p, nbins):
        pltpu.sync_copy(hloc.at[pl.ds(0, nbins)], shr.at[s, p, pl.ds(0, nbins)])
        plsc.subcore_barrier()
        for t in range(4):
            pltpu.sync_copy(shr.at[g + t, p, pl.ds(0, nbins)],
                            t4.at[t, pl.ds(0, nbins)])

        def csum(i, _):
            hcmb[dsv(i)] = (t4[0, dsv(i)] + t4[1, dsv(i)]
                            + t4[2, dsv(i)] + t4[3, dsv(i)])
            return 0

        lax.fori_loop(0, nbins // 16, csum, 0)

    def scan(nbins, target):
        def body(i, carry):
            run, p, c_at, h_at = carry
            h16 = hcmb[dsv(i)]
            cum = plsc.cumsum(h16) + run
            cross = cum > target
            npop = jnp.max(plsc.all_reduce_population_count(cross))
            ffs = jnp.max(plsc.all_reduce_ffs(cross))
            oh = lane == ffs
            cv = jnp.max(jnp.where(oh, cum, 0))
            hv = jnp.max(jnp.where(oh, h16, 0))
            isnew = (p < 0) & (npop > 0)
            p = jnp.where(isnew, i * 16 + ffs, p)
            c_at = jnp.where(isnew, cv, c_at)
            h_at = jnp.where(isnew, hv, h_at)
            return jnp.max(cum), p, c_at, h_at

        _, p, c_at, h_at = lax.fori_loop(
            0, nbins // 16, body,
            (jnp.int32(0), jnp.int32(-1), jnp.int32(0), jnp.int32(0)))
        return p, c_at, h_at

    hist_pass(4096, lambda b: _lsr(b, 20), None)
    combine(0, 4096)
    p1, c1, h1 = scan(4096, jnp.int32(HW - K))
    k2 = jnp.int32(K) - (jnp.int32(HW) - c1)

    hist_pass(4096, lambda b: _lsr(b, 8) & 0xFFF,
              lambda b: _lsr(b, 20) == p1)
    combine(1, 4096)
    p2, c2, h2 = scan(4096, h1 - k2)
    k3 = k2 - (h1 - c2)

    hist_pass(256, lambda b: b & 0xFF,
              lambda b: _lsr(b, 8) == ((p1 << 12) | p2))
    combine(2, 256)
    p3, c3, h3 = scan(256, h2 - k3)
    k4 = k3 - (h2 - c3)

    t_key = ((p1 << 20) | (p2 << 8) | p3) ^ imin

    my16 = hloc[pl.ds(pl.multiple_of((p3 // 16) * 16, 16), 16)]
    myeq = jnp.max(jnp.where(lane == (p3 % 16), my16, 0))
    hloc[pl.ds(0, 16)] = jnp.broadcast_to(myeq, (16,)).astype(jnp.int32)
    pltpu.sync_copy(hloc.at[pl.ds(0, 16)], shr.at[s, 3, pl.ds(0, 16)])
    plsc.subcore_barrier()
    neq_before = jnp.int32(0)
    for t in range(3):
        pltpu.sync_copy(shr.at[g + t, 3, pl.ds(0, 16)], t4.at[t, pl.ds(0, 16)])
        cnt_t = jnp.max(t4[t, pl.ds(0, 16)])
        neq_before += jnp.where(jnp.int32(t) < q, cnt_t, 0)
    need_local = k4 - neq_before

    def mask_build(i, r):
        key = kbuf[dsv(i)]
        gt = key > t_key
        eq = key == t_key
        ec = jnp.where(eq, 1, 0).astype(jnp.int32)
        cum = plsc.cumsum(ec)
        take = eq & ((r + cum - ec) < need_local)
        ebuf[dsv(i)] = jnp.where(gt | take, 1.0, 0.0).astype(jnp.float32)
        return r + jnp.max(cum)

    lax.fori_loop(0, NVQ, mask_build, jnp.int32(0))
    pltpu.sync_copy(ebuf, m_hbm.at[pl.ds(base, Q)])


_sc_select = functools.partial(
    pl.kernel,
    out_type=jax.ShapeDtypeStruct((B * HW,), jnp.float32),
    mesh=plsc.VectorSubcoreMesh(core_axis_name="c", subcore_axis_name="s",
                                num_cores=2, num_subcores=16),
    compiler_params=pltpu.CompilerParams(needs_layout_passes=False),
    scratch_types=[
        pltpu.VMEM((Q,), jnp.float32),
        pltpu.VMEM((Q,), jnp.int32),
        pltpu.VMEM((4096,), jnp.int32),
        pltpu.VMEM((4096,), jnp.int32),
        pltpu.VMEM((4, 4096), jnp.int32),
        pltpu.VMEM_SHARED((16, 4, 4096), jnp.int32),
    ],
)(_sc_select_body)


def _apply_body(x_ref, m_ref, o_ref):
    o_ref[...] = x_ref[...] * m_ref[...][:, None]


@jax.jit
def kernel(x):
    xr = x.reshape(B, C, SL, 128)

    energy = pl.pallas_call(
        _mean_body,
        grid=(B, C // CB_MEAN),
        in_specs=[pl.BlockSpec((1, CB_MEAN, SL, 128), lambda b, c: (b, c, 0, 0))],
        out_specs=pl.BlockSpec((1, SL, 128), lambda b, c: (b, 0, 0)),
        out_shape=jax.ShapeDtypeStruct((B, SL, 128), jnp.float32),
    )(xr)

    mask = _sc_select(energy.reshape(B * HW)).reshape(B, SL, 128)

    out = pl.pallas_call(
        _apply_body,
        grid=(B, SL // TS_APPLY),
        in_specs=[
            pl.BlockSpec((1, C, TS_APPLY, 128), lambda b, j: (b, 0, j, 0)),
            pl.BlockSpec((1, TS_APPLY, 128), lambda b, j: (b, j, 0)),
        ],
        out_specs=pl.BlockSpec((1, C, TS_APPLY, 128), lambda b, j: (b, 0, j, 0)),
        out_shape=jax.ShapeDtypeStruct((B, C, SL, 128), jnp.float32),
    )(xr, mask)

    return out.reshape(B, C, H, W)


# speedup vs baseline: 1.0486x; 1.0011x over previous
import functools

import jax
import jax.numpy as jnp
from jax import lax
from jax.experimental import pallas as pl
from jax.experimental.pallas import tpu as pltpu
from jax.experimental.pallas import tpu_sc as plsc

B, C, H, W = 8, 96, 384, 384
HW = H * W
SL = HW // 128
K = HW // 2
CB_MEAN = 16
TS_APPLY = 192
I32MIN = -(2**31)
NQ = 4
Q = HW // NQ
NVQ = Q // 16


def _mean_body(x_ref, o_ref):
    c = pl.program_id(1)

    @pl.when(c == 0)
    def _init():
        o_ref[...] = jnp.zeros_like(o_ref)

    o_ref[...] += jnp.sum(x_ref[...], axis=1)

    @pl.when(c == (C // CB_MEAN) - 1)
    def _fin():
        o_ref[...] *= jnp.float32(1.0 / C)


def _lsr(x, n):
    return lax.shift_right_logical(x, jnp.int32(n) if isinstance(n, int) else n)


def _sc_select_body(e_hbm, m_hbm, ebuf, kbuf, hloc, hcmb, t4, shr):
    c = lax.axis_index("c")
    s = lax.axis_index("s")
    row = c * 4 + s // 4
    q = s % 4
    g = s - q
    base = row * HW + q * Q
    imin = jnp.int32(I32MIN)
    ones = jnp.ones((16,), jnp.int32)
    lane = jax.lax.broadcasted_iota(jnp.int32, (16,), 0)

    def dsv(i):
        return pl.ds(pl.multiple_of(i * 16, 16), 16)

    pltpu.sync_copy(e_hbm.at[pl.ds(base, Q)], ebuf)

    def keys_init(i, _):
        u = lax.bitcast_convert_type(ebuf[dsv(i)], jnp.int32)
        kbuf[dsv(i)] = jnp.where(u >= 0, u, imin - u)
        return 0

    lax.fori_loop(0, NVQ, keys_init, 0)

    def hist_pass(nbins, bin_fn, match_fn):
        def zero(i, _):
            hloc[dsv(i)] = jnp.zeros((16,), jnp.int32)
            return 0

        lax.fori_loop(0, nbins // 16, zero, 0)

        def acc(i, _):
            b = kbuf[dsv(i)] ^ imin
            if match_fn is None:
                plsc.addupdate_scatter(hloc, [bin_fn(b)], ones)
            else:
                plsc.addupdate_scatter(hloc, [bin_fn(b)], ones,
                                       mask=match_fn(b))
            return 0

        lax.fori_loop(0, NVQ, acc, 0)

    def combine(p, nbins):
        pltpu.sync_copy(hloc.at[pl.ds(0, nbins)], shr.at[s, p, pl.ds(0, nbins)])
        plsc.subcore_barrier()
        for t in range(4):
            pltpu.sync_copy(shr.at[g + t, p, pl.ds(0, nbins)],
                            t4.at[t, pl.ds(0, nbins)])

        def csum(i, _):
            hcmb[dsv(i)] = (t4[0, dsv(i)] + t4[1, dsv(i)]
                            + t4[2, dsv(i)] + t4[3, dsv(i)])
            return 0

        lax.fori_loop(0, nbins // 16, csum, 0)

    def scan(nbins, target):
        def body(i, carry):
            run, p, c_at, h_at = carry
            h16 = hcmb[dsv(i)]
            cum = plsc.cumsum(h16) + run
            cross = cum > target
            npop = jnp.max(plsc.all_reduce_population_count(cross))
            ffs = jnp.max(plsc.all_reduce_ffs(cross))
            oh = lane == ffs
            cv = jnp.max(jnp.where(oh, cum, 0))
            hv = jnp.max(jnp.where(oh, h16, 0))
            isnew = (p < 0) & (npop > 0)
            p = jnp.where(isnew, i * 16 + ffs, p)
            c_at = jnp.where(isnew, cv, c_at)
            h_at = jnp.where(isnew, hv, h_at)
            return jnp.max(cum), p, c_at, h_at

        _, p, c_at, h_at = lax.fori_loop(
            0, nbins // 16, body,
            (jnp.int32(0), jnp.int32(-1), jnp.int32(0), jnp.int32(0)))
        return p, c_at, h_at

    hist_pass(4096, lambda b: _lsr(b, 20), None)
    combine(0, 4096)
    p1, c1, h1 = scan(4096, jnp.int32(HW - K))
    k2 = jnp.int32(K) - (jnp.int32(HW) - c1)

    hist_pass(4096, lambda b: _lsr(b, 8) & 0xFFF,
              lambda b: _lsr(b, 20) == p1)
    combine(1, 4096)
    p2, c2, h2 = scan(4096, h1 - k2)
    k3 = k2 - (h1 - c2)

    hist_pass(256, lambda b: b & 0xFF,
              lambda b: _lsr(b, 8) == ((p1 << 12) | p2))
    combine(2, 256)
    p3, c3, h3 = scan(256, h2 - k3)
    k4 = k3 - (h2 - c3)

    t_key = ((p1 << 20) | (p2 << 8) | p3) ^ imin

    my16 = hloc[pl.ds(pl.multiple_of((p3 // 16) * 16, 16), 16)]
    myeq = jnp.max(jnp.where(lane == (p3 % 16), my16, 0))
    hloc[pl.ds(0, 16)] = jnp.broadcast_to(myeq, (16,)).astype(jnp.int32)
    pltpu.sync_copy(hloc.at[pl.ds(0, 16)], shr.at[s, 3, pl.ds(0, 16)])
    plsc.subcore_barrier()
    neq_before = jnp.int32(0)
    for t in range(3):
        pltpu.sync_copy(shr.at[g + t, 3, pl.ds(0, 16)], t4.at[t, pl.ds(0, 16)])
        cnt_t = jnp.max(t4[t, pl.ds(0, 16)])
        neq_before += jnp.where(jnp.int32(t) < q, cnt_t, 0)
    need_local = k4 - neq_before

    def mask_build(i, r):
        key = kbuf[dsv(i)]
        gt = key > t_key
        eq = key == t_key
        ec = jnp.where(eq, 1, 0).astype(jnp.int32)
        cum = plsc.cumsum(ec)
        take = eq & ((r + cum - ec) < need_local)
        ebuf[dsv(i)] = jnp.where(gt | take, 1.0, 0.0).astype(jnp.float32)
        return r + jnp.max(cum)

    lax.fori_loop(0, NVQ, mask_build, jnp.int32(0))
    pltpu.sync_copy(ebuf, m_hbm.at[pl.ds(base, Q)])


_sc_select = functools.partial(
    pl.kernel,
    out_type=jax.ShapeDtypeStruct((B * HW,), jnp.float32),
    mesh=plsc.VectorSubcoreMesh(core_axis_name="c", subcore_axis_name="s",
                                num_cores=2, num_subcores=16),
    compiler_params=pltpu.CompilerParams(needs_layout_passes=False),
    scratch_types=[
        pltpu.VMEM((Q,), jnp.float32),
        pltpu.VMEM((Q,), jnp.int32),
        pltpu.VMEM((4096,), jnp.int32),
        pltpu.VMEM((4096,), jnp.int32),
        pltpu.VMEM((4, 4096), jnp.int32),
        pltpu.VMEM_SHARED((16, 4, 4096), jnp.int32),
    ],
)(_sc_select_body)


def _apply_body(x_ref, m_ref, o_ref):
    o_ref[...] = x_ref[...] * m_ref[...][:, None]


@jax.jit
def kernel(x):
    xr = x.reshape(B, C, SL, 128)

    energy = pl.pallas_call(
        _mean_body,
        grid=(B, C // CB_MEAN),
        in_specs=[pl.BlockSpec((1, CB_MEAN, SL, 128), lambda b, c: (b, c, 0, 0))],
        out_specs=pl.BlockSpec((1, SL, 128), lambda b, c: (b, 0, 0)),
        out_shape=jax.ShapeDtypeStruct((B, SL, 128), jnp.float32),
    )(xr)

    mask = _sc_select(energy.reshape(B * HW)).reshape(B, SL, 128)

    out = pl.pallas_call(
        _apply_body,
        grid=(B, SL // TS_APPLY),
        in_specs=[
            pl.BlockSpec((1, C, TS_APPLY, 128), lambda b, j: (b, 0, j, 0)),
            pl.BlockSpec((1, TS_APPLY, 128), lambda b, j: (b, j, 0)),
        ],
        out_specs=pl.BlockSpec((1, C, TS_APPLY, 128), lambda b, j: (b, 0, j, 0)),
        out_shape=jax.ShapeDtypeStruct((B, C, SL, 128), jnp.float32),
    )(xr, mask)

    return out.reshape(B, C, H, W)


# speedup vs baseline: 1.0627x; 1.0134x over previous
import functools

import jax
import jax.numpy as jnp
from jax import lax
from jax.experimental import pallas as pl
from jax.experimental.pallas import tpu as pltpu
from jax.experimental.pallas import tpu_sc as plsc

B, C, H, W = 8, 96, 384, 384
HW = H * W
SL = HW // 128
K = HW // 2
CB_MEAN = 16
TS_APPLY = 192
I32MIN = -(2**31)
SC_ROWS = 4
TPR = 32 // SC_ROWS
Q = HW // TPR
NVQ = Q // 16


def _mean_body(x_ref, o_ref):
    c = pl.program_id(1)

    @pl.when(c == 0)
    def _init():
        o_ref[...] = jnp.zeros_like(o_ref)

    o_ref[...] += jnp.sum(x_ref[...], axis=1)

    @pl.when(c == (C // CB_MEAN) - 1)
    def _fin():
        o_ref[...] *= jnp.float32(1.0 / C)


def _lsr(x, n):
    return lax.shift_right_logical(x, jnp.int32(n) if isinstance(n, int) else n)


def _sc_select_body(e_hbm, m_hbm, ebuf, kbuf, hloc, hcmb, t4, shr):
    c = lax.axis_index("c")
    s = lax.axis_index("s")
    row = c * (SC_ROWS // 2) + s // TPR
    q = s % TPR
    g = s - q
    base = row * HW + q * Q
    imin = jnp.int32(I32MIN)
    ones = jnp.ones((16,), jnp.int32)
    lane = jax.lax.broadcasted_iota(jnp.int32, (16,), 0)

    def dsv(i):
        return pl.ds(pl.multiple_of(i * 16, 16), 16)

    pltpu.sync_copy(e_hbm.at[pl.ds(base, Q)], ebuf)

    def keys_init(i, _):
        u = lax.bitcast_convert_type(ebuf[dsv(i)], jnp.int32)
        kbuf[dsv(i)] = jnp.where(u >= 0, u, imin - u)
        return 0

    lax.fori_loop(0, NVQ, keys_init, 0)

    def hist_pass(nbins, bin_fn, match_fn):
        def zero(i, _):
            hloc[dsv(i)] = jnp.zeros((16,), jnp.int32)
            return 0

        lax.fori_loop(0, nbins // 16, zero, 0)

        def acc(i, _):
            b = kbuf[dsv(i)] ^ imin
            if match_fn is None:
                plsc.addupdate_scatter(hloc, [bin_fn(b)], ones)
            else:
                plsc.addupdate_scatter(hloc, [bin_fn(b)], ones,
                                       mask=match_fn(b))
            return 0

        lax.fori_loop(0, NVQ, acc, 0)

    def combine(p, nbins):
        pltpu.sync_copy(hloc.at[pl.ds(0, nbins)], shr.at[s, p, pl.ds(0, nbins)])
        plsc.subcore_barrier()
        for t in range(TPR):
            pltpu.sync_copy(shr.at[g + t, p, pl.ds(0, nbins)],
                            t4.at[t, pl.ds(0, nbins)])

        def csum(i, _):
            acc = t4[0, dsv(i)]
            for t in range(1, TPR):
                acc = acc + t4[t, dsv(i)]
            hcmb[dsv(i)] = acc
            return 0

        lax.fori_loop(0, nbins // 16, csum, 0)

    def scan(nbins, target):
        def body(i, carry):
            run, p, c_at, h_at = carry
            h16 = hcmb[dsv(i)]
            cum = plsc.cumsum(h16) + run
            cross = cum > target
            npop = jnp.max(plsc.all_reduce_population_count(cross))
            ffs = jnp.max(plsc.all_reduce_ffs(cross))
            oh = lane == ffs
            cv = jnp.max(jnp.where(oh, cum, 0))
            hv = jnp.max(jnp.where(oh, h16, 0))
            isnew = (p < 0) & (npop > 0)
            p = jnp.where(isnew, i * 16 + ffs, p)
            c_at = jnp.where(isnew, cv, c_at)
            h_at = jnp.where(isnew, hv, h_at)
            return jnp.max(cum), p, c_at, h_at

        _, p, c_at, h_at = lax.fori_loop(
            0, nbins // 16, body,
            (jnp.int32(0), jnp.int32(-1), jnp.int32(0), jnp.int32(0)))
        return p, c_at, h_at

    hist_pass(4096, lambda b: _lsr(b, 20), None)
    combine(0, 4096)
    p1, c1, h1 = scan(4096, jnp.int32(HW - K))
    k2 = jnp.int32(K) - (jnp.int32(HW) - c1)

    hist_pass(4096, lambda b: _lsr(b, 8) & 0xFFF,
              lambda b: _lsr(b, 20) == p1)
    combine(1, 4096)
    p2, c2, h2 = scan(4096, h1 - k2)
    k3 = k2 - (h1 - c2)

    hist_pass(256, lambda b: b & 0xFF,
              lambda b: _lsr(b, 8) == ((p1 << 12) | p2))
    combine(2, 256)
    p3, c3, h3 = scan(256, h2 - k3)
    k4 = k3 - (h2 - c3)

    t_key = ((p1 << 20) | (p2 << 8) | p3) ^ imin

    my16 = hloc[pl.ds(pl.multiple_of((p3 // 16) * 16, 16), 16)]
    myeq = jnp.max(jnp.where(lane == (p3 % 16), my16, 0))
    hloc[pl.ds(0, 16)] = jnp.broadcast_to(myeq, (16,)).astype(jnp.int32)
    pltpu.sync_copy(hloc.at[pl.ds(0, 16)], shr.at[s, 3, pl.ds(0, 16)])
    plsc.subcore_barrier()
    neq_before = jnp.int32(0)
    for t in range(TPR - 1):
        pltpu.sync_copy(shr.at[g + t, 3, pl.ds(0, 16)], t4.at[t, pl.ds(0, 16)])
        cnt_t = jnp.max(t4[t, pl.ds(0, 16)])
        neq_before += jnp.where(jnp.int32(t) < q, cnt_t, 0)
    need_local = k4 - neq_before

    def mask_build(i, r):
        key = kbuf[dsv(i)]
        gt = key > t_key
        eq = key == t_key
        ec = jnp.where(eq, 1, 0).astype(jnp.int32)
        cum = plsc.cumsum(ec)
        take = eq & ((r + cum - ec) < need_local)
        ebuf[dsv(i)] = jnp.where(gt | take, 1.0, 0.0).astype(jnp.float32)
        return r + jnp.max(cum)

    lax.fori_loop(0, NVQ, mask_build, jnp.int32(0))
    pltpu.sync_copy(ebuf, m_hbm.at[pl.ds(base, Q)])


_sc_select = functools.partial(
    pl.kernel,
    out_type=jax.ShapeDtypeStruct((SC_ROWS * HW,), jnp.float32),
    mesh=plsc.VectorSubcoreMesh(core_axis_name="c", subcore_axis_name="s",
                                num_cores=2, num_subcores=16),
    compiler_params=pltpu.CompilerParams(needs_layout_passes=False),
    scratch_types=[
        pltpu.VMEM((Q,), jnp.float32),
        pltpu.VMEM((Q,), jnp.int32),
        pltpu.VMEM((4096,), jnp.int32),
        pltpu.VMEM((4096,), jnp.int32),
        pltpu.VMEM((TPR, 4096), jnp.int32),
        pltpu.VMEM_SHARED((16, 4, 4096), jnp.int32),
    ],
)(_sc_select_body)


def _apply_body(x_ref, m_ref, o_ref):
    o_ref[...] = x_ref[...] * m_ref[...][:, None]


@jax.jit
def kernel(x):
    xr = x.reshape(B, C, SL, 128)

    def mean_half(off):
        return pl.pallas_call(
            _mean_body,
            grid=(B // 2, C // CB_MEAN),
            in_specs=[pl.BlockSpec((1, CB_MEAN, SL, 128),
                                   lambda b, c: (b + off, c, 0, 0))],
            out_specs=pl.BlockSpec((1, SL, 128), lambda b, c: (b, 0, 0)),
            out_shape=jax.ShapeDtypeStruct((B // 2, SL, 128), jnp.float32),
        )(xr)

    e1 = mean_half(0)
    m1 = _sc_select(e1.reshape(SC_ROWS * HW))
    e2 = mean_half(B // 2)
    m2 = _sc_select(e2.reshape(SC_ROWS * HW))
    mask = jnp.concatenate(
        [m1.reshape(B // 2, SL, 128), m2.reshape(B // 2, SL, 128)])

    out = pl.pallas_call(
        _apply_body,
        grid=(B, SL // TS_APPLY),
        in_specs=[
            pl.BlockSpec((1, C, TS_APPLY, 128), lambda b, j: (b, 0, j, 0)),
            pl.BlockSpec((1, TS_APPLY, 128), lambda b, j: (b, j, 0)),
        ],
        out_specs=pl.BlockSpec((1, C, TS_APPLY, 128), lambda b, j: (b, 0, j, 0)),
        out_shape=jax.ShapeDtypeStruct((B, C, SL, 128), jnp.float32),
    )(xr, mask)

    return out.reshape(B, C, H, W)
